# baseline (speedup 1.0000x reference)
import functools

import jax
import jax.numpy as jnp
from jax import lax
from jax.experimental import pallas as pl
from jax.experimental.pallas import tpu as pltpu
from jax.experimental.pallas import tpu_sc as plsc

N = 10000
NP = 10240
D = 128
HID = 256
E = 320000
K = 5000
NC, NS = 2, 16
NW = NC * NS
CH = 128
NCH = 80
EW = NCH * CH
EPAD = NW * EW
RP = NP // NS
NROW = 80

_MESH = plsc.VectorSubcoreMesh(
    core_axis_name="c", subcore_axis_name="s", num_cores=NC, num_subcores=NS)

def _z16():
    return jnp.zeros((16,), jnp.float32)


def _o16():
    return jnp.ones((16,), jnp.float32)


def _deg_body(srcw, dstw, out, idx_s, idx_d, ones_v, zb, acc_o, acc_i):
    c = lax.axis_index("c")
    s = lax.axis_index("s")
    wid = s * NC + c
    for t in range(RP // 16):
        zb[pl.ds(t * 16, 16)] = _z16()
    for t in range(CH // 16):
        ones_v[pl.ds(t * 16, 16)] = _o16()
    pltpu.sync_copy(zb, acc_o.at[pl.ds(s * RP, RP)])
    pltpu.sync_copy(zb, acc_i.at[pl.ds(s * RP, RP)])
    plsc.subcore_barrier()
    pltpu.sync_copy(srcw.at[wid], idx_s)
    pltpu.sync_copy(dstw.at[wid], idx_d)

    def body(ch, _):
        pltpu.sync_copy(ones_v, acc_o.at[idx_s.at[ch]], add=True)
        pltpu.sync_copy(ones_v, acc_i.at[idx_d.at[ch]], add=True)
        return 0

    lax.fori_loop(0, NCH, body, 0)
    plsc.subcore_barrier()
    pltpu.sync_copy(acc_o.at[pl.ds(s * RP, RP)], out.at[c, 0, pl.ds(s * RP, RP)])
    pltpu.sync_copy(acc_i.at[pl.ds(s * RP, RP)], out.at[c, 1, pl.ds(s * RP, RP)])


_deg_call = functools.partial(
    pl.kernel,
    out_type=jax.ShapeDtypeStruct((NC, 2, NP), jnp.float32),
    mesh=_MESH,
    scratch_types=[
        pltpu.VMEM((NCH, CH), jnp.int32),
        pltpu.VMEM((NCH, CH), jnp.int32),
        pltpu.VMEM((CH,), jnp.float32),
        pltpu.VMEM((RP,), jnp.float32),
        pltpu.VMEM_SHARED((NP,), jnp.float32),
        pltpu.VMEM_SHARED((NP,), jnp.float32),
    ],
)(_deg_body)


def _vecagg_body(table, combw, out, comb, ib_s, ib_d, rows0, rows1, zb, acc,
                 sem0, sem1):
    c = lax.axis_index("c")
    s = lax.axis_index("s")
    wid = s * NC + c
    for r in range(8):
        for t in range(D // 16):
            zb[r, pl.ds(t * 16, 16)] = _z16()

    def zloop(j, _):
        pltpu.sync_copy(zb, acc.at[pl.ds(s * RP + j * 8, 8)])
        return 0

    lax.fori_loop(0, RP // 8, zloop, 0)
    plsc.subcore_barrier()
    pltpu.sync_copy(combw.at[wid], comb)

    def unpack(ch, slot):
        for t in range(CH // 16):
            cv = comb[ch, pl.ds(t * 16, 16)]
            ib_s[slot, pl.ds(t * 16, 16)] = cv & jnp.int32(0xFFFF)
            ib_d[slot, pl.ds(t * 16, 16)] = lax.shift_right_logical(
                cv, jnp.int32(16))

    unpack(0, 0)
    pltpu.async_copy(table.at[ib_s.at[0]], rows0, sem0)
    unpack(1, 1)
    pltpu.async_copy(table.at[ib_s.at[1]], rows1, sem1)

    def body(j, _):
        a = 2 * j
        pltpu.make_async_copy(table.at[ib_s.at[0]], rows0, sem0).wait()
        pltpu.sync_copy(rows0, acc.at[ib_d.at[0]], add=True)

        @pl.when(a + 2 < NCH)
        def _():
            unpack(a + 2, 0)
            pltpu.async_copy(table.at[ib_s.at[0]], rows0, sem0)

        pltpu.make_async_copy(table.at[ib_s.at[1]], rows1, sem1).wait()
        pltpu.sync_copy(rows1, acc.at[ib_d.at[1]], add=True)

        @pl.when(a + 3 < NCH)
        def _():
            unpack(a + 3, 1)
            pltpu.async_copy(table.at[ib_s.at[1]], rows1, sem1)

        return 0

    lax.fori_loop(0, NCH // 2, body, 0)
    plsc.subcore_barrier()
    pltpu.sync_copy(acc.at[pl.ds(s * RP, RP)], out.at[c, pl.ds(s * RP, RP)])


_vecagg_call = functools.partial(
    pl.kernel,
    out_type=jax.ShapeDtypeStruct((NC, NP, D), jnp.float32),
    mesh=_MESH,
    scratch_types=[
        pltpu.VMEM((NCH, CH), jnp.int32),
        pltpu.VMEM((2, CH), jnp.int32),
        pltpu.VMEM((2, CH), jnp.int32),
        pltpu.VMEM((CH, D), jnp.float32),
        pltpu.VMEM((CH, D), jnp.float32),
        pltpu.VMEM((8, D), jnp.float32),
        pltpu.VMEM_SHARED((NP, D), jnp.float32),
        pltpu.SemaphoreType.DMA,
        pltpu.SemaphoreType.DMA,
    ],
)(_vecagg_body)


def _scalagg_body(table, srcw, dstw, out, idx_s, idx_d, vals, zb, acc, sem):
    c = lax.axis_index("c")
    s = lax.axis_index("s")
    wid = s * NC + c
    for t in range(RP // 16):
        zb[pl.ds(t * 16, 16)] = _z16()
    pltpu.sync_copy(zb, acc.at[pl.ds(s * RP, RP)])
    plsc.subcore_barrier()
    pltpu.sync_copy(srcw.at[wid], idx_s)
    pltpu.sync_copy(dstw.at[wid], idx_d)

    def body(ch, _):
        pltpu.async_copy(table.at[idx_s.at[ch]], vals, sem).wait()
        pltpu.sync_copy(vals, acc.at[idx_d.at[ch]], add=True)
        return 0

    lax.fori_loop(0, NCH, body, 0)
    plsc.subcore_barrier()
    pltpu.sync_copy(acc.at[pl.ds(s * RP, RP)], out.at[c, pl.ds(s * RP, RP)])


_scalagg_call = functools.partial(
    pl.kernel,
    out_type=jax.ShapeDtypeStruct((NC, NP), jnp.float32),
    mesh=_MESH,
    scratch_types=[
        pltpu.VMEM((NCH, CH), jnp.int32),
        pltpu.VMEM((NCH, CH), jnp.int32),
        pltpu.VMEM((CH,), jnp.float32),
        pltpu.VMEM((RP,), jnp.float32),
        pltpu.VMEM_SHARED((NP,), jnp.float32),
        pltpu.SemaphoreType.DMA,
    ],
)(_scalagg_body)


def _prep_body(dp_ref, so_ref, si_ref):
    dp = dp_ref[...]
    deg_o = dp[0, 0] + dp[1, 0]
    deg_i = dp[0, 1] + dp[1, 1]
    r = (lax.broadcasted_iota(jnp.int32, (NROW, 128), 0) * 128
         + lax.broadcasted_iota(jnp.int32, (NROW, 128), 1))
    valid = r < N
    so_ref[...] = jnp.where(valid, lax.rsqrt(jnp.maximum(deg_o, 1.0)), 0.0)
    si_ref[...] = jnp.where(valid, lax.rsqrt(jnp.maximum(deg_i, 1.0)), 0.0)


def _prep_call(degp):
    return pl.pallas_call(
        _prep_body,
        out_shape=[jax.ShapeDtypeStruct((NROW, 128), jnp.float32),
                   jax.ShapeDtypeStruct((NROW, 128), jnp.float32)],
    )(degp)


_BR = 2048
_NB = NP // _BR


def _scale_body(x_ref, m_ref, o_ref):
    o_ref[...] = x_ref[...] * m_ref[...]


def _scale_call(xp, m_col):
    return pl.pallas_call(
        _scale_body,
        grid=(_NB,),
        in_specs=[pl.BlockSpec((_BR, D), lambda b: (b, 0)),
                  pl.BlockSpec((_BR, 1), lambda b: (b, 0))],
        out_specs=pl.BlockSpec((_BR, D), lambda b: (b, 0)),
        out_shape=jax.ShapeDtypeStruct((NP, D), jnp.float32),
    )(xp, m_col)


def _mid_body(p_ref, si_ref, so_ref, o_ref):
    o_ref[...] = (p_ref[0] + p_ref[1]) * (si_ref[...] * so_ref[...])


def _mid_call(p, si_col, so_col):
    return pl.pallas_call(
        _mid_body,
        grid=(_NB,),
        in_specs=[pl.BlockSpec((NC, _BR, D), lambda b: (0, b, 0)),
                  pl.BlockSpec((_BR, 1), lambda b: (b, 0)),
                  pl.BlockSpec((_BR, 1), lambda b: (b, 0))],
        out_specs=pl.BlockSpec((_BR, D), lambda b: (b, 0)),
        out_shape=jax.ShapeDtypeStruct((NP, D), jnp.float32),
    )(p, si_col, so_col)


def _wts_body(w0_ref, w1_ref, w2_ref, w01_ref, w012_ref):
    w01 = jnp.dot(w0_ref[...], w1_ref[...], preferred_element_type=jnp.float32)
    w01_ref[...] = w01
    w012_ref[...] = jnp.dot(w01, w2_ref[...], preferred_element_type=jnp.float32)


def _wts_call(W0, W1, W2):
    return pl.pallas_call(
        _wts_body,
        out_shape=[jax.ShapeDtypeStruct((D, HID), jnp.float32),
                   jax.ShapeDtypeStruct((D, HID), jnp.float32)],
    )(W0, W1, W2)


def _feats_body(p1_ref, p2_ref, p3_ref, si_ref, so_ref, w0_ref, w01_ref,
                w012_ref, b0_ref, b1_ref, b2_ref, wsa_ref, wsb_ref, wsc_ref,
                cat_ref, s0n_ref):
    si = si_ref[...]
    y1 = (p1_ref[0] + p1_ref[1]) * si
    y2 = (p2_ref[0] + p2_ref[1]) * si
    y3 = (p3_ref[0] + p3_ref[1]) * si
    f1 = jnp.dot(y1, w0_ref[...], preferred_element_type=jnp.float32) + b0_ref[...]
    f2 = jnp.dot(y2, w01_ref[...], preferred_element_type=jnp.float32) + b1_ref[...]
    f3 = jnp.dot(y3, w012_ref[...], preferred_element_type=jnp.float32) + b2_ref[...]
    cat_ref[:, 0:HID] = f1
    cat_ref[:, HID:2 * HID] = f2
    cat_ref[:, 2 * HID:3 * HID] = f3
    s0 = (jnp.dot(f1, wsa_ref[...], preferred_element_type=jnp.float32)
          + jnp.dot(f2, wsb_ref[...], preferred_element_type=jnp.float32)
          + jnp.dot(f3, wsc_ref[...], preferred_element_type=jnp.float32))
    s0n_ref[...] = s0 * so_ref[...]


def _feats_call(p1, p2, p3, si_col, so_col, W0, W01, W012, b0, b1, b2,
                wsa, wsb, wsc):
    pspec = pl.BlockSpec((NC, _BR, D), lambda b: (0, b, 0))
    cspec = pl.BlockSpec((_BR, 1), lambda b: (b, 0))
    wspec = pl.BlockSpec((D, HID), lambda b: (0, 0))
    bspec = pl.BlockSpec((1, HID), lambda b: (0, 0))
    sspec = pl.BlockSpec((HID, 1), lambda b: (0, 0))
    return pl.pallas_call(
        _feats_body,
        grid=(_NB,),
        in_specs=[pspec, pspec, pspec, cspec, cspec, wspec, wspec, wspec,
                  bspec, bspec, bspec, sspec, sspec, sspec],
        out_specs=[pl.BlockSpec((_BR, 3 * HID), lambda b: (b, 0)),
                   pl.BlockSpec((_BR, 1), lambda b: (b, 0))],
        out_shape=[jax.ShapeDtypeStruct((NP, 3 * HID), jnp.float32),
                   jax.ShapeDtypeStruct((NP, 1), jnp.float32)],
    )(p1, p2, p3, si_col, so_col, W0, W01, W012, b0, b1, b2, wsa, wsb, wsc)


_MIN32 = -2147483648


def _e0_body(sp_ref, si_ref, bs_ref, w_ref, m_ref):
    sc = (sp_ref[0] + sp_ref[1]) * si_ref[...] + bs_ref[0, 0]
    r = (lax.broadcasted_iota(jnp.int32, (NROW, 128), 0) * 128
         + lax.broadcasted_iota(jnp.int32, (NROW, 128), 1))
    sc = jnp.where(r < N, sc, -jnp.inf)
    b = lax.bitcast_convert_type(sc, jnp.int32)
    m = b ^ (lax.shift_right_arithmetic(b, 31) & jnp.int32(0x7FFFFFFF))

    def tbody(i, t):
        cand = t | lax.shift_left(jnp.int32(1), 31 - i)
        thr_s = cand ^ jnp.int32(_MIN32)
        cnt = jnp.sum((m >= thr_s).astype(jnp.int32))
        return jnp.where(cnt >= K, cand, t)

    T = lax.fori_loop(0, 32, tbody, jnp.int32(0))
    ts = T ^ jnp.int32(_MIN32)
    c_gt = jnp.sum((m > ts).astype(jnp.int32))
    need = K - c_gt
    ties = m == ts

    def jbody(i, jv):
        cand = jv | lax.shift_left(jnp.int32(1), 13 - i)
        cnt = jnp.sum((ties & (r < cand)).astype(jnp.int32))
        return jnp.where(cnt < need, cand, jv)

    J = lax.fori_loop(0, 14, jbody, jnp.int32(0))
    sel = (m > ts) | (ties & (r <= J) & (need > 0))
    g = jnp.tanh(sc)
    w_ref[...] = jnp.where(sel, g, 0.0)
    m_ref[...] = jnp.where(sel, 1.0, 0.0)


def _e0_call(sp, si80, bs11):
    return pl.pallas_call(
        _e0_body,
        out_shape=[jax.ShapeDtypeStruct((NROW, 128), jnp.float32),
                   jax.ShapeDtypeStruct((NROW, 128), jnp.float32)],
    )(sp, si80, bs11)


def _e1_body(w_ref, m_ref, cat_ref, wl1_ref, bl1_ref, wl2_ref, bl2_ref,
             wl3_ref, bl3_ref, out_ref, sum_acc, max_acc):
    b = pl.program_id(0)

    @pl.when(b < _NB)
    def _():
        gated = cat_ref[...] * w_ref[...]
        psum = jnp.sum(gated, axis=0, keepdims=True)
        pmax = jnp.max(jnp.where(m_ref[...] > 0.0, gated, -jnp.inf),
                       axis=0, keepdims=True)

        @pl.when(b == 0)
        def _():
            sum_acc[...] = psum
            max_acc[...] = pmax

        @pl.when(b > 0)
        def _():
            sum_acc[...] = sum_acc[...] + psum
            max_acc[...] = jnp.maximum(max_acc[...], pmax)

    @pl.when(b == _NB)
    def _():
        mean = sum_acc[...] * (1.0 / K)
        ro = jnp.concatenate([mean, max_acc[...]], axis=1)
        h = jnp.maximum(
            jnp.dot(ro, wl1_ref[...], preferred_element_type=jnp.float32)
            + bl1_ref[...], 0.0)
        h = jnp.maximum(
            jnp.dot(h, wl2_ref[...], preferred_element_type=jnp.float32)
            + bl2_ref[...], 0.0)
        lg = (jnp.dot(h, wl3_ref[...], preferred_element_type=jnp.float32)
              + bl3_ref[...])
        mx = jnp.max(lg, axis=1, keepdims=True)
        lse = jnp.log(jnp.sum(jnp.exp(lg - mx), axis=1, keepdims=True)) + mx
        out_ref[...] = lg - lse


def _e1_call(w_col, m_col, cat, Wl1, bl1, Wl2, bl2, Wl3, bl3):
    clamp = lambda b: (jnp.minimum(b, _NB - 1), 0)
    return pl.pallas_call(
        _e1_body,
        grid=(_NB + 1,),
        in_specs=[pl.BlockSpec((_BR, 1), clamp),
                  pl.BlockSpec((_BR, 1), clamp),
                  pl.BlockSpec((_BR, 3 * HID), clamp),
                  pl.BlockSpec((2 * 3 * HID, HID), lambda b: (0, 0)),
                  pl.BlockSpec((1, HID), lambda b: (0, 0)),
                  pl.BlockSpec((HID, HID // 2), lambda b: (0, 0)),
                  pl.BlockSpec((1, HID // 2), lambda b: (0, 0)),
                  pl.BlockSpec((HID // 2, 10), lambda b: (0, 0)),
                  pl.BlockSpec((1, 10), lambda b: (0, 0))],
        out_specs=pl.BlockSpec((1, 10), lambda b: (0, 0)),
        out_shape=jax.ShapeDtypeStruct((1, 10), jnp.float32),
        scratch_shapes=[pltpu.VMEM((1, 3 * HID), jnp.float32),
                        pltpu.VMEM((1, 3 * HID), jnp.float32)],
    )(w_col, m_col, cat, Wl1, bl1, Wl2, bl2, Wl3, bl3)


def kernel(x, edge_index, W0, b0, W1, b1, W2, b2, Ws, bs,
           Wl1, bl1, Wl2, bl2, Wl3, bl3):
    src = edge_index[0]
    dst = edge_index[1]
    pad = EPAD - E
    fill = jnp.full((pad,), NP - 1, jnp.int32)
    srcp = jnp.concatenate([src, fill]).reshape(NW, NCH, CH)
    dstp = jnp.concatenate([dst, fill]).reshape(NW, NCH, CH)
    xp = jnp.concatenate([x, jnp.zeros((NP - N, D), x.dtype)], axis=0)

    degp = _deg_call(srcp, dstp)
    so80, si80 = _prep_call(degp.reshape(NC, 2, NROW, 128))
    so_col = so80.reshape(NP, 1)
    si_col = si80.reshape(NP, 1)

    combw = srcp | (dstp << 16)
    xn = _scale_call(xp, so_col)
    p1 = _vecagg_call(xn, combw)
    y1n = _mid_call(p1, si_col, so_col)
    p2 = _vecagg_call(y1n, combw)
    y2n = _mid_call(p2, si_col, so_col)
    p3 = _vecagg_call(y2n, combw)

    W01, W012 = _wts_call(W0, W1, W2)
    cat, s0n = _feats_call(
        p1, p2, p3, si_col, so_col, W0, W01, W012,
        b0.reshape(1, HID), b1.reshape(1, HID), b2.reshape(1, HID),
        Ws[0:HID], Ws[HID:2 * HID], Ws[2 * HID:3 * HID])

    sp = _scalagg_call(s0n.reshape(NP), srcp, dstp)
    w80, m80 = _e0_call(sp.reshape(NC, NROW, 128), si80, bs.reshape(1, 1))
    return _e1_call(w80.reshape(NP, 1), m80.reshape(NP, 1), cat,
                    Wl1, bl1.reshape(1, HID), Wl2, bl2.reshape(1, HID // 2),
                    Wl3, bl3.reshape(1, 10))


# speedup vs baseline: 1.2335x; 1.2335x over previous
import functools

import jax
import jax.numpy as jnp
from jax import lax
from jax.experimental import pallas as pl
from jax.experimental.pallas import tpu as pltpu
from jax.experimental.pallas import tpu_sc as plsc

N = 10000
NP = 10240
D = 128
HID = 256
E = 320000
K = 5000
NC, NS = 2, 16
NW = NC * NS
CH = 128
NCH = 80
VCH = 64
VNCH = 160
EW = NCH * CH
EPAD = NW * EW
RP = NP // NS
NROW = 80

_MESH = plsc.VectorSubcoreMesh(
    core_axis_name="c", subcore_axis_name="s", num_cores=NC, num_subcores=NS)

def _z16():
    return jnp.zeros((16,), jnp.float32)


def _o16():
    return jnp.ones((16,), jnp.float32)


def _deg_body(srcw, dstw, out, idx_s, idx_d, ones_v, zb, acc_o, acc_i):
    c = lax.axis_index("c")
    s = lax.axis_index("s")
    wid = s * NC + c
    for t in range(RP // 16):
        zb[pl.ds(t * 16, 16)] = _z16()
    for t in range(CH // 16):
        ones_v[pl.ds(t * 16, 16)] = _o16()
    pltpu.sync_copy(zb, acc_o.at[pl.ds(s * RP, RP)])
    pltpu.sync_copy(zb, acc_i.at[pl.ds(s * RP, RP)])
    plsc.subcore_barrier()
    pltpu.sync_copy(srcw.at[wid], idx_s)
    pltpu.sync_copy(dstw.at[wid], idx_d)

    def body(ch, _):
        pltpu.sync_copy(ones_v, acc_o.at[idx_s.at[ch]], add=True)
        pltpu.sync_copy(ones_v, acc_i.at[idx_d.at[ch]], add=True)
        return 0

    lax.fori_loop(0, NCH, body, 0)
    plsc.subcore_barrier()
    pltpu.sync_copy(acc_o.at[pl.ds(s * RP, RP)], out.at[c, 0, pl.ds(s * RP, RP)])
    pltpu.sync_copy(acc_i.at[pl.ds(s * RP, RP)], out.at[c, 1, pl.ds(s * RP, RP)])


_deg_call = functools.partial(
    pl.kernel,
    out_type=jax.ShapeDtypeStruct((NC, 2, NP), jnp.float32),
    mesh=_MESH,
    scratch_types=[
        pltpu.VMEM((NCH, CH), jnp.int32),
        pltpu.VMEM((NCH, CH), jnp.int32),
        pltpu.VMEM((CH,), jnp.float32),
        pltpu.VMEM((RP,), jnp.float32),
        pltpu.VMEM_SHARED((NP,), jnp.float32),
        pltpu.VMEM_SHARED((NP,), jnp.float32),
    ],
)(_deg_body)


def _vecagg_body(table, combw, out, comb, ib_s, ib_d, rows0, rows1, rows2,
                 rows3, zb, acc, gs0, gs1, gs2, gs3, ss0, ss1, ss2, ss3):
    c = lax.axis_index("c")
    s = lax.axis_index("s")
    wid = s * NC + c
    for r in range(8):
        for t in range(D // 16):
            zb[r, pl.ds(t * 16, 16)] = _z16()

    def zloop(j, _):
        pltpu.sync_copy(zb, acc.at[pl.ds(s * RP + j * 8, 8)])
        return 0

    lax.fori_loop(0, RP // 8, zloop, 0)
    plsc.subcore_barrier()
    pltpu.sync_copy(combw.at[wid], comb)

    rows = (rows0, rows1, rows2, rows3)
    gsem = (gs0, gs1, gs2, gs3)
    ssem = (ss0, ss1, ss2, ss3)

    def unpack(row, half, slot):
        for t in range(VCH // 16):
            cv = comb[row, pl.ds(half * VCH + t * 16, 16)]
            ib_s[slot, pl.ds(t * 16, 16)] = cv & jnp.int32(0xFFFF)
            ib_d[slot, pl.ds(t * 16, 16)] = lax.shift_right_logical(
                cv, jnp.int32(16))

    def gather(row, half, slot):
        unpack(row, half, slot)
        pltpu.async_copy(table.at[ib_s.at[slot]], rows[slot], gsem[slot])

    def wait_g_scat(slot):
        pltpu.make_async_copy(table.at[ib_s.at[slot]], rows[slot],
                              gsem[slot]).wait()
        pltpu.async_copy(rows[slot], acc.at[ib_d.at[slot]], ssem[slot],
                         add=True)

    def wait_s(slot):
        pltpu.make_async_copy(rows[slot], acc.at[ib_d.at[slot]],
                              ssem[slot]).wait()

    for p in range(4):
        gather(p >> 1, p & 1, p)
    for p in range(4):
        wait_g_scat(p)

    def body(j, _):
        for p in range(4):
            wait_s(p)
            gather(2 * j + (p >> 1), p & 1, p)
        for p in range(4):
            wait_g_scat(p)
        return 0

    lax.fori_loop(1, VNCH // 4, body, 0)
    for p in range(4):
        wait_s(p)
    plsc.subcore_barrier()
    pltpu.sync_copy(acc.at[pl.ds(s * RP, RP)], out.at[c, pl.ds(s * RP, RP)])


_vecagg_call = functools.partial(
    pl.kernel,
    out_type=jax.ShapeDtypeStruct((NC, NP, D), jnp.float32),
    mesh=_MESH,
    scratch_types=[
        pltpu.VMEM((NCH, CH), jnp.int32),
        pltpu.VMEM((4, VCH), jnp.int32),
        pltpu.VMEM((4, VCH), jnp.int32),
        pltpu.VMEM((VCH, D), jnp.float32),
        pltpu.VMEM((VCH, D), jnp.float32),
        pltpu.VMEM((VCH, D), jnp.float32),
        pltpu.VMEM((VCH, D), jnp.float32),
        pltpu.VMEM((8, D), jnp.float32),
        pltpu.VMEM_SHARED((NP, D), jnp.float32),
        pltpu.SemaphoreType.DMA,
        pltpu.SemaphoreType.DMA,
        pltpu.SemaphoreType.DMA,
        pltpu.SemaphoreType.DMA,
        pltpu.SemaphoreType.DMA,
        pltpu.SemaphoreType.DMA,
        pltpu.SemaphoreType.DMA,
        pltpu.SemaphoreType.DMA,
    ],
)(_vecagg_body)


def _scalagg_body(table, srcw, dstw, out, idx_s, idx_d, vals, zb, acc,
                  gs0, gs1, gs2, gs3, ss0, ss1, ss2, ss3):
    c = lax.axis_index("c")
    s = lax.axis_index("s")
    wid = s * NC + c
    for t in range(RP // 16):
        zb[pl.ds(t * 16, 16)] = _z16()
    pltpu.sync_copy(zb, acc.at[pl.ds(s * RP, RP)])
    plsc.subcore_barrier()
    pltpu.sync_copy(srcw.at[wid], idx_s)
    pltpu.sync_copy(dstw.at[wid], idx_d)

    gsem = (gs0, gs1, gs2, gs3)
    ssem = (ss0, ss1, ss2, ss3)

    def gather(ch, p):
        pltpu.async_copy(table.at[idx_s.at[ch]], vals.at[p], gsem[p])

    def wait_g_scat(ch, p):
        pltpu.make_async_copy(table.at[idx_s.at[ch]], vals.at[p],
                              gsem[p]).wait()
        pltpu.async_copy(vals.at[p], acc.at[idx_d.at[ch]], ssem[p], add=True)

    def wait_s(ch, p):
        pltpu.make_async_copy(vals.at[p], acc.at[idx_d.at[ch]],
                              ssem[p]).wait()

    for p in range(4):
        gather(p, p)
    for p in range(4):
        wait_g_scat(p, p)

    def body(j, _):
        a = 4 * j
        for p in range(4):
            wait_s(a - 4 + p, p)
            gather(a + p, p)
        for p in range(4):
            wait_g_scat(a + p, p)
        return 0

    lax.fori_loop(1, NCH // 4, body, 0)
    for p in range(4):
        wait_s(NCH - 4 + p, p)
    plsc.subcore_barrier()
    pltpu.sync_copy(acc.at[pl.ds(s * RP, RP)], out.at[c, pl.ds(s * RP, RP)])


_scalagg_call = functools.partial(
    pl.kernel,
    out_type=jax.ShapeDtypeStruct((NC, NP), jnp.float32),
    mesh=_MESH,
    scratch_types=[
        pltpu.VMEM((NCH, CH), jnp.int32),
        pltpu.VMEM((NCH, CH), jnp.int32),
        pltpu.VMEM((4, CH), jnp.float32),
        pltpu.VMEM((RP,), jnp.float32),
        pltpu.VMEM_SHARED((NP,), jnp.float32),
        pltpu.SemaphoreType.DMA,
        pltpu.SemaphoreType.DMA,
        pltpu.SemaphoreType.DMA,
        pltpu.SemaphoreType.DMA,
        pltpu.SemaphoreType.DMA,
        pltpu.SemaphoreType.DMA,
        pltpu.SemaphoreType.DMA,
        pltpu.SemaphoreType.DMA,
    ],
)(_scalagg_body)


def _prep_body(dp_ref, so_ref, si_ref):
    dp = dp_ref[...]
    deg_o = dp[0, 0] + dp[1, 0]
    deg_i = dp[0, 1] + dp[1, 1]
    r = (lax.broadcasted_iota(jnp.int32, (NROW, 128), 0) * 128
         + lax.broadcasted_iota(jnp.int32, (NROW, 128), 1))
    valid = r < N
    so_ref[...] = jnp.where(valid, lax.rsqrt(jnp.maximum(deg_o, 1.0)), 0.0)
    si_ref[...] = jnp.where(valid, lax.rsqrt(jnp.maximum(deg_i, 1.0)), 0.0)


def _prep_call(degp):
    return pl.pallas_call(
        _prep_body,
        out_shape=[jax.ShapeDtypeStruct((NROW, 128), jnp.float32),
                   jax.ShapeDtypeStruct((NROW, 128), jnp.float32)],
    )(degp)


_BR = 2048
_NB = NP // _BR


def _scale_body(x_ref, m_ref, o_ref):
    o_ref[...] = x_ref[...] * m_ref[...]


def _scale_call(xp, m_col):
    return pl.pallas_call(
        _scale_body,
        grid=(_NB,),
        in_specs=[pl.BlockSpec((_BR, D), lambda b: (b, 0)),
                  pl.BlockSpec((_BR, 1), lambda b: (b, 0))],
        out_specs=pl.BlockSpec((_BR, D), lambda b: (b, 0)),
        out_shape=jax.ShapeDtypeStruct((NP, D), jnp.float32),
    )(xp, m_col)


def _mid_body(p_ref, si_ref, so_ref, o_ref):
    o_ref[...] = (p_ref[0] + p_ref[1]) * (si_ref[...] * so_ref[...])


def _mid_call(p, si_col, so_col):
    return pl.pallas_call(
        _mid_body,
        grid=(_NB,),
        in_specs=[pl.BlockSpec((NC, _BR, D), lambda b: (0, b, 0)),
                  pl.BlockSpec((_BR, 1), lambda b: (b, 0)),
                  pl.BlockSpec((_BR, 1), lambda b: (b, 0))],
        out_specs=pl.BlockSpec((_BR, D), lambda b: (b, 0)),
        out_shape=jax.ShapeDtypeStruct((NP, D), jnp.float32),
    )(p, si_col, so_col)


def _wts_body(w0_ref, w1_ref, w2_ref, w01_ref, w012_ref):
    w01 = jnp.dot(w0_ref[...], w1_ref[...], preferred_element_type=jnp.float32)
    w01_ref[...] = w01
    w012_ref[...] = jnp.dot(w01, w2_ref[...], preferred_element_type=jnp.float32)


def _wts_call(W0, W1, W2):
    return pl.pallas_call(
        _wts_body,
        out_shape=[jax.ShapeDtypeStruct((D, HID), jnp.float32),
                   jax.ShapeDtypeStruct((D, HID), jnp.float32)],
    )(W0, W1, W2)


def _feats_body(p1_ref, p2_ref, p3_ref, si_ref, so_ref, w0_ref, w01_ref,
                w012_ref, b0_ref, b1_ref, b2_ref, wsa_ref, wsb_ref, wsc_ref,
                cat_ref, s0n_ref):
    si = si_ref[...]
    y1 = (p1_ref[0] + p1_ref[1]) * si
    y2 = (p2_ref[0] + p2_ref[1]) * si
    y3 = (p3_ref[0] + p3_ref[1]) * si
    f1 = jnp.dot(y1, w0_ref[...], preferred_element_type=jnp.float32) + b0_ref[...]
    f2 = jnp.dot(y2, w01_ref[...], preferred_element_type=jnp.float32) + b1_ref[...]
    f3 = jnp.dot(y3, w012_ref[...], preferred_element_type=jnp.float32) + b2_ref[...]
    cat_ref[:, 0:HID] = f1
    cat_ref[:, HID:2 * HID] = f2
    cat_ref[:, 2 * HID:3 * HID] = f3
    s0 = (jnp.dot(f1, wsa_ref[...], preferred_element_type=jnp.float32)
          + jnp.dot(f2, wsb_ref[...], preferred_element_type=jnp.float32)
          + jnp.dot(f3, wsc_ref[...], preferred_element_type=jnp.float32))
    s0n_ref[...] = s0 * so_ref[...]


def _feats_call(p1, p2, p3, si_col, so_col, W0, W01, W012, b0, b1, b2,
                wsa, wsb, wsc):
    pspec = pl.BlockSpec((NC, _BR, D), lambda b: (0, b, 0))
    cspec = pl.BlockSpec((_BR, 1), lambda b: (b, 0))
    wspec = pl.BlockSpec((D, HID), lambda b: (0, 0))
    bspec = pl.BlockSpec((1, HID), lambda b: (0, 0))
    sspec = pl.BlockSpec((HID, 1), lambda b: (0, 0))
    return pl.pallas_call(
        _feats_body,
        grid=(_NB,),
        in_specs=[pspec, pspec, pspec, cspec, cspec, wspec, wspec, wspec,
                  bspec, bspec, bspec, sspec, sspec, sspec],
        out_specs=[pl.BlockSpec((_BR, 3 * HID), lambda b: (b, 0)),
                   pl.BlockSpec((_BR, 1), lambda b: (b, 0))],
        out_shape=[jax.ShapeDtypeStruct((NP, 3 * HID), jnp.float32),
                   jax.ShapeDtypeStruct((NP, 1), jnp.float32)],
    )(p1, p2, p3, si_col, so_col, W0, W01, W012, b0, b1, b2, wsa, wsb, wsc)


_MIN32 = -2147483648


def _e0_body(sp_ref, si_ref, bs_ref, w_ref, m_ref):
    sc = (sp_ref[0] + sp_ref[1]) * si_ref[...] + bs_ref[0, 0]
    r = (lax.broadcasted_iota(jnp.int32, (NROW, 128), 0) * 128
         + lax.broadcasted_iota(jnp.int32, (NROW, 128), 1))
    sc = jnp.where(r < N, sc, -jnp.inf)
    b = lax.bitcast_convert_type(sc, jnp.int32)
    m = b ^ (lax.shift_right_arithmetic(b, 31) & jnp.int32(0x7FFFFFFF))

    def tbody(i, t):
        cand = t | lax.shift_left(jnp.int32(1), 31 - i)
        thr_s = cand ^ jnp.int32(_MIN32)
        cnt = jnp.sum((m >= thr_s).astype(jnp.int32))
        return jnp.where(cnt >= K, cand, t)

    T = lax.fori_loop(0, 32, tbody, jnp.int32(0))
    ts = T ^ jnp.int32(_MIN32)
    c_gt = jnp.sum((m > ts).astype(jnp.int32))
    need = K - c_gt
    ties = m == ts

    def jbody(i, jv):
        cand = jv | lax.shift_left(jnp.int32(1), 13 - i)
        cnt = jnp.sum((ties & (r < cand)).astype(jnp.int32))
        return jnp.where(cnt < need, cand, jv)

    J = lax.fori_loop(0, 14, jbody, jnp.int32(0))
    sel = (m > ts) | (ties & (r <= J) & (need > 0))
    g = jnp.tanh(sc)
    w_ref[...] = jnp.where(sel, g, 0.0)
    m_ref[...] = jnp.where(sel, 1.0, 0.0)


def _e0_call(sp, si80, bs11):
    return pl.pallas_call(
        _e0_body,
        out_shape=[jax.ShapeDtypeStruct((NROW, 128), jnp.float32),
                   jax.ShapeDtypeStruct((NROW, 128), jnp.float32)],
    )(sp, si80, bs11)


def _e1_body(w_ref, m_ref, cat_ref, wl1_ref, bl1_ref, wl2_ref, bl2_ref,
             wl3_ref, bl3_ref, out_ref, sum_acc, max_acc):
    b = pl.program_id(0)

    @pl.when(b < _NB)
    def _():
        gated = cat_ref[...] * w_ref[...]
        psum = jnp.sum(gated, axis=0, keepdims=True)
        pmax = jnp.max(jnp.where(m_ref[...] > 0.0, gated, -jnp.inf),
                       axis=0, keepdims=True)

        @pl.when(b == 0)
        def _():
            sum_acc[...] = psum
            max_acc[...] = pmax

        @pl.when(b > 0)
        def _():
            sum_acc[...] = sum_acc[...] + psum
            max_acc[...] = jnp.maximum(max_acc[...], pmax)

    @pl.when(b == _NB)
    def _():
        mean = sum_acc[...] * (1.0 / K)
        ro = jnp.concatenate([mean, max_acc[...]], axis=1)
        h = jnp.maximum(
            jnp.dot(ro, wl1_ref[...], preferred_element_type=jnp.float32)
            + bl1_ref[...], 0.0)
        h = jnp.maximum(
            jnp.dot(h, wl2_ref[...], preferred_element_type=jnp.float32)
            + bl2_ref[...], 0.0)
        lg = (jnp.dot(h, wl3_ref[...], preferred_element_type=jnp.float32)
              + bl3_ref[...])
        mx = jnp.max(lg, axis=1, keepdims=True)
        lse = jnp.log(jnp.sum(jnp.exp(lg - mx), axis=1, keepdims=True)) + mx
        out_ref[...] = lg - lse


def _e1_call(w_col, m_col, cat, Wl1, bl1, Wl2, bl2, Wl3, bl3):
    clamp = lambda b: (jnp.minimum(b, _NB - 1), 0)
    return pl.pallas_call(
        _e1_body,
        grid=(_NB + 1,),
        in_specs=[pl.BlockSpec((_BR, 1), clamp),
                  pl.BlockSpec((_BR, 1), clamp),
                  pl.BlockSpec((_BR, 3 * HID), clamp),
                  pl.BlockSpec((2 * 3 * HID, HID), lambda b: (0, 0)),
                  pl.BlockSpec((1, HID), lambda b: (0, 0)),
                  pl.BlockSpec((HID, HID // 2), lambda b: (0, 0)),
                  pl.BlockSpec((1, HID // 2), lambda b: (0, 0)),
                  pl.BlockSpec((HID // 2, 10), lambda b: (0, 0)),
                  pl.BlockSpec((1, 10), lambda b: (0, 0))],
        out_specs=pl.BlockSpec((1, 10), lambda b: (0, 0)),
        out_shape=jax.ShapeDtypeStruct((1, 10), jnp.float32),
        scratch_shapes=[pltpu.VMEM((1, 3 * HID), jnp.float32),
                        pltpu.VMEM((1, 3 * HID), jnp.float32)],
    )(w_col, m_col, cat, Wl1, bl1, Wl2, bl2, Wl3, bl3)


def kernel(x, edge_index, W0, b0, W1, b1, W2, b2, Ws, bs,
           Wl1, bl1, Wl2, bl2, Wl3, bl3):
    src = edge_index[0]
    dst = edge_index[1]
    pad = EPAD - E
    fill = jnp.full((pad,), NP - 1, jnp.int32)
    srcp = jnp.concatenate([src, fill]).reshape(NW, NCH, CH)
    dstp = jnp.concatenate([dst, fill]).reshape(NW, NCH, CH)
    xp = jnp.concatenate([x, jnp.zeros((NP - N, D), x.dtype)], axis=0)

    degp = _deg_call(srcp, dstp)
    so80, si80 = _prep_call(degp.reshape(NC, 2, NROW, 128))
    so_col = so80.reshape(NP, 1)
    si_col = si80.reshape(NP, 1)

    combw = srcp | (dstp << 16)
    xn = _scale_call(xp, so_col)
    p1 = _vecagg_call(xn, combw)
    y1n = _mid_call(p1, si_col, so_col)
    p2 = _vecagg_call(y1n, combw)
    y2n = _mid_call(p2, si_col, so_col)
    p3 = _vecagg_call(y2n, combw)

    W01, W012 = _wts_call(W0, W1, W2)
    cat, s0n = _feats_call(
        p1, p2, p3, si_col, so_col, W0, W01, W012,
        b0.reshape(1, HID), b1.reshape(1, HID), b2.reshape(1, HID),
        Ws[0:HID], Ws[HID:2 * HID], Ws[2 * HID:3 * HID])

    sp = _scalagg_call(s0n.reshape(NP), srcp, dstp)
    w80, m80 = _e0_call(sp.reshape(NC, NROW, 128), si80, bs.reshape(1, 1))
    return _e1_call(w80.reshape(NP, 1), m80.reshape(NP, 1), cat,
                    Wl1, bl1.reshape(1, HID), Wl2, bl2.reshape(1, HID // 2),
                    Wl3, bl3.reshape(1, 10))


# speedup vs baseline: 1.3006x; 1.0544x over previous
import functools

import jax
import jax.numpy as jnp
from jax import lax
from jax.experimental import pallas as pl
from jax.experimental.pallas import tpu as pltpu
from jax.experimental.pallas import tpu_sc as plsc

N = 10000
NP = 10240
D = 128
HID = 256
E = 320000
K = 5000
NC, NS = 2, 16
NW = NC * NS
CH = 128
NCH = 80
ACH = 120
BCH = 40
NPA = 10112
RPA = NPA // NS
EW = NCH * CH
EPAD = NW * EW
RP = NP // NS
NROW = 80

_MESH = plsc.VectorSubcoreMesh(
    core_axis_name="c", subcore_axis_name="s", num_cores=NC, num_subcores=NS)

def _z16():
    return jnp.zeros((16,), jnp.float32)


def _o16():
    return jnp.ones((16,), jnp.float32)


def _deg_body(srcw, dstw, out, idx_s, idx_d, ones_v, zb, acc_o, acc_i):
    c = lax.axis_index("c")
    s = lax.axis_index("s")
    wid = s * NC + c
    for t in range(RP // 16):
        zb[pl.ds(t * 16, 16)] = _z16()
    for t in range(CH // 16):
        ones_v[pl.ds(t * 16, 16)] = _o16()
    pltpu.sync_copy(zb, acc_o.at[pl.ds(s * RP, RP)])
    pltpu.sync_copy(zb, acc_i.at[pl.ds(s * RP, RP)])
    plsc.subcore_barrier()
    pltpu.sync_copy(srcw.at[wid], idx_s)
    pltpu.sync_copy(dstw.at[wid], idx_d)

    def body(ch, _):
        pltpu.sync_copy(ones_v, acc_o.at[idx_s.at[ch]], add=True)
        pltpu.sync_copy(ones_v, acc_i.at[idx_d.at[ch]], add=True)
        return 0

    lax.fori_loop(0, NCH, body, 0)
    plsc.subcore_barrier()
    pltpu.sync_copy(acc_o.at[pl.ds(s * RP, RP)], out.at[c, 0, pl.ds(s * RP, RP)])
    pltpu.sync_copy(acc_i.at[pl.ds(s * RP, RP)], out.at[c, 1, pl.ds(s * RP, RP)])


_deg_call = functools.partial(
    pl.kernel,
    out_type=jax.ShapeDtypeStruct((NC, 2, NP), jnp.float32),
    mesh=_MESH,
    scratch_types=[
        pltpu.VMEM((NCH, CH), jnp.int32),
        pltpu.VMEM((NCH, CH), jnp.int32),
        pltpu.VMEM((CH,), jnp.float32),
        pltpu.VMEM((RP,), jnp.float32),
        pltpu.VMEM_SHARED((NP,), jnp.float32),
        pltpu.VMEM_SHARED((NP,), jnp.float32),
    ],
)(_deg_body)


def _vecagg_body(table, combw, ztab, out, comb, ib_s, ib_d, rows0, rows1,
                 acc, gs0, gs1, ss0, ss1):
    c = lax.axis_index("c")
    s = lax.axis_index("s")
    wid = s * NC + c
    nch = jnp.where(c == 0, ACH, BCH)
    pltpu.sync_copy(ztab, acc.at[pl.ds(s * RPA, RPA)])
    plsc.subcore_barrier()
    pltpu.sync_copy(combw.at[wid], comb)

    rows = (rows0, rows1)
    gsem = (gs0, gs1)
    ssem = (ss0, ss1)

    def unpack(ch, slot):
        for t in range(CH // 16):
            cv = comb[ch, pl.ds(t * 16, 16)]
            ib_s[slot, pl.ds(t * 16, 16)] = cv & jnp.int32(0xFFFF)
            ib_d[slot, pl.ds(t * 16, 16)] = lax.shift_right_logical(
                cv, jnp.int32(16))

    def gather(ch, slot):
        unpack(ch, slot)
        pltpu.async_copy(table.at[ib_s.at[slot]], rows[slot], gsem[slot])

    def wait_g_scat(slot):
        pltpu.make_async_copy(table.at[ib_s.at[slot]], rows[slot],
                              gsem[slot]).wait()
        pltpu.async_copy(rows[slot], acc.at[ib_d.at[slot]], ssem[slot],
                         add=True)

    def wait_s(slot):
        pltpu.make_async_copy(rows[slot], acc.at[ib_d.at[slot]],
                              ssem[slot]).wait()

    for p in range(2):
        gather(p, p)
    for p in range(2):
        wait_g_scat(p)

    def body(j, _):
        a = 2 * j
        for p in range(2):
            wait_s(p)
            gather(a + p, p)
        for p in range(2):
            wait_g_scat(p)
        return 0

    lax.fori_loop(1, nch // 2, body, 0)
    for p in range(2):
        wait_s(p)
    plsc.subcore_barrier()
    pltpu.sync_copy(acc.at[pl.ds(s * RPA, RPA)],
                    out.at[c, pl.ds(s * RPA, RPA)])


_vecagg_call = functools.partial(
    pl.kernel,
    out_type=jax.ShapeDtypeStruct((NC, NP, D), jnp.float32),
    mesh=_MESH,
    scratch_types=[
        pltpu.VMEM((ACH, CH), jnp.int32),
        pltpu.VMEM((2, CH), jnp.int32),
        pltpu.VMEM((2, CH), jnp.int32),
        pltpu.VMEM((CH, D), jnp.float32),
        pltpu.VMEM((CH, D), jnp.float32),
        pltpu.VMEM_SHARED((NPA, D), jnp.float32),
        pltpu.SemaphoreType.DMA,
        pltpu.SemaphoreType.DMA,
        pltpu.SemaphoreType.DMA,
        pltpu.SemaphoreType.DMA,
    ],
)(_vecagg_body)


def _scalagg_body(table, srcw, dstw, out, idx_s, idx_d, vals, zb, acc,
                  gs0, gs1, gs2, gs3, ss0, ss1, ss2, ss3):
    c = lax.axis_index("c")
    s = lax.axis_index("s")
    wid = s * NC + c
    for t in range(RP // 16):
        zb[pl.ds(t * 16, 16)] = _z16()
    pltpu.sync_copy(zb, acc.at[pl.ds(s * RP, RP)])
    plsc.subcore_barrier()
    pltpu.sync_copy(srcw.at[wid], idx_s)
    pltpu.sync_copy(dstw.at[wid], idx_d)

    gsem = (gs0, gs1, gs2, gs3)
    ssem = (ss0, ss1, ss2, ss3)

    def gather(ch, p):
        pltpu.async_copy(table.at[idx_s.at[ch]], vals.at[p], gsem[p])

    def wait_g_scat(ch, p):
        pltpu.make_async_copy(table.at[idx_s.at[ch]], vals.at[p],
                              gsem[p]).wait()
        pltpu.async_copy(vals.at[p], acc.at[idx_d.at[ch]], ssem[p], add=True)

    def wait_s(ch, p):
        pltpu.make_async_copy(vals.at[p], acc.at[idx_d.at[ch]],
                              ssem[p]).wait()

    for p in range(4):
        gather(p, p)
    for p in range(4):
        wait_g_scat(p, p)

    def body(j, _):
        a = 4 * j
        for p in range(4):
            wait_s(a - 4 + p, p)
            gather(a + p, p)
        for p in range(4):
            wait_g_scat(a + p, p)
        return 0

    lax.fori_loop(1, NCH // 4, body, 0)
    for p in range(4):
        wait_s(NCH - 4 + p, p)
    plsc.subcore_barrier()
    pltpu.sync_copy(acc.at[pl.ds(s * RP, RP)], out.at[c, pl.ds(s * RP, RP)])


_scalagg_call = functools.partial(
    pl.kernel,
    out_type=jax.ShapeDtypeStruct((NC, NP), jnp.float32),
    mesh=_MESH,
    scratch_types=[
        pltpu.VMEM((NCH, CH), jnp.int32),
        pltpu.VMEM((NCH, CH), jnp.int32),
        pltpu.VMEM((4, CH), jnp.float32),
        pltpu.VMEM((RP,), jnp.float32),
        pltpu.VMEM_SHARED((NP,), jnp.float32),
        pltpu.SemaphoreType.DMA,
        pltpu.SemaphoreType.DMA,
        pltpu.SemaphoreType.DMA,
        pltpu.SemaphoreType.DMA,
        pltpu.SemaphoreType.DMA,
        pltpu.SemaphoreType.DMA,
        pltpu.SemaphoreType.DMA,
        pltpu.SemaphoreType.DMA,
    ],
)(_scalagg_body)


def _prep_body(dp_ref, so_ref, si_ref):
    dp = dp_ref[...]
    deg_o = dp[0, 0] + dp[1, 0]
    deg_i = dp[0, 1] + dp[1, 1]
    r = (lax.broadcasted_iota(jnp.int32, (NROW, 128), 0) * 128
         + lax.broadcasted_iota(jnp.int32, (NROW, 128), 1))
    valid = r < N
    so_ref[...] = jnp.where(valid, lax.rsqrt(jnp.maximum(deg_o, 1.0)), 0.0)
    si_ref[...] = jnp.where(valid, lax.rsqrt(jnp.maximum(deg_i, 1.0)), 0.0)


def _prep_call(degp):
    return pl.pallas_call(
        _prep_body,
        out_shape=[jax.ShapeDtypeStruct((NROW, 128), jnp.float32),
                   jax.ShapeDtypeStruct((NROW, 128), jnp.float32)],
    )(degp)


_BR = 2048
_NB = NP // _BR


def _scale_body(x_ref, m_ref, o_ref):
    o_ref[...] = x_ref[...] * m_ref[...]


def _scale_call(xp, m_col):
    return pl.pallas_call(
        _scale_body,
        grid=(_NB,),
        in_specs=[pl.BlockSpec((_BR, D), lambda b: (b, 0)),
                  pl.BlockSpec((_BR, 1), lambda b: (b, 0))],
        out_specs=pl.BlockSpec((_BR, D), lambda b: (b, 0)),
        out_shape=jax.ShapeDtypeStruct((NP, D), jnp.float32),
    )(xp, m_col)


def _mid_body(p_ref, si_ref, so_ref, o_ref):
    o_ref[...] = (p_ref[0] + p_ref[1]) * (si_ref[...] * so_ref[...])


def _mid_call(p, si_col, so_col):
    return pl.pallas_call(
        _mid_body,
        grid=(_NB,),
        in_specs=[pl.BlockSpec((NC, _BR, D), lambda b: (0, b, 0)),
                  pl.BlockSpec((_BR, 1), lambda b: (b, 0)),
                  pl.BlockSpec((_BR, 1), lambda b: (b, 0))],
        out_specs=pl.BlockSpec((_BR, D), lambda b: (b, 0)),
        out_shape=jax.ShapeDtypeStruct((NP, D), jnp.float32),
    )(p, si_col, so_col)


def _wts_body(w0_ref, w1_ref, w2_ref, w01_ref, w012_ref):
    w01 = jnp.dot(w0_ref[...], w1_ref[...], preferred_element_type=jnp.float32)
    w01_ref[...] = w01
    w012_ref[...] = jnp.dot(w01, w2_ref[...], preferred_element_type=jnp.float32)


def _wts_call(W0, W1, W2):
    return pl.pallas_call(
        _wts_body,
        out_shape=[jax.ShapeDtypeStruct((D, HID), jnp.float32),
                   jax.ShapeDtypeStruct((D, HID), jnp.float32)],
    )(W0, W1, W2)


def _feats_body(p1_ref, p2_ref, p3_ref, si_ref, so_ref, w0_ref, w01_ref,
                w012_ref, b0_ref, b1_ref, b2_ref, wsa_ref, wsb_ref, wsc_ref,
                cat_ref, s0n_ref):
    si = si_ref[...]
    y1 = (p1_ref[0] + p1_ref[1]) * si
    y2 = (p2_ref[0] + p2_ref[1]) * si
    y3 = (p3_ref[0] + p3_ref[1]) * si
    f1 = jnp.dot(y1, w0_ref[...], preferred_element_type=jnp.float32) + b0_ref[...]
    f2 = jnp.dot(y2, w01_ref[...], preferred_element_type=jnp.float32) + b1_ref[...]
    f3 = jnp.dot(y3, w012_ref[...], preferred_element_type=jnp.float32) + b2_ref[...]
    cat_ref[:, 0:HID] = f1
    cat_ref[:, HID:2 * HID] = f2
    cat_ref[:, 2 * HID:3 * HID] = f3
    s0 = (jnp.dot(f1, wsa_ref[...], preferred_element_type=jnp.float32)
          + jnp.dot(f2, wsb_ref[...], preferred_element_type=jnp.float32)
          + jnp.dot(f3, wsc_ref[...], preferred_element_type=jnp.float32))
    s0n_ref[...] = s0 * so_ref[...]


def _feats_call(p1, p2, p3, si_col, so_col, W0, W01, W012, b0, b1, b2,
                wsa, wsb, wsc):
    pspec = pl.BlockSpec((NC, _BR, D), lambda b: (0, b, 0))
    cspec = pl.BlockSpec((_BR, 1), lambda b: (b, 0))
    wspec = pl.BlockSpec((D, HID), lambda b: (0, 0))
    bspec = pl.BlockSpec((1, HID), lambda b: (0, 0))
    sspec = pl.BlockSpec((HID, 1), lambda b: (0, 0))
    return pl.pallas_call(
        _feats_body,
        grid=(_NB,),
        in_specs=[pspec, pspec, pspec, cspec, cspec, wspec, wspec, wspec,
                  bspec, bspec, bspec, sspec, sspec, sspec],
        out_specs=[pl.BlockSpec((_BR, 3 * HID), lambda b: (b, 0)),
                   pl.BlockSpec((_BR, 1), lambda b: (b, 0))],
        out_shape=[jax.ShapeDtypeStruct((NP, 3 * HID), jnp.float32),
                   jax.ShapeDtypeStruct((NP, 1), jnp.float32)],
    )(p1, p2, p3, si_col, so_col, W0, W01, W012, b0, b1, b2, wsa, wsb, wsc)


_MIN32 = -2147483648


def _e0_body(sp_ref, si_ref, bs_ref, w_ref, m_ref):
    sc = (sp_ref[0] + sp_ref[1]) * si_ref[...] + bs_ref[0, 0]
    r = (lax.broadcasted_iota(jnp.int32, (NROW, 128), 0) * 128
         + lax.broadcasted_iota(jnp.int32, (NROW, 128), 1))
    sc = jnp.where(r < N, sc, -jnp.inf)
    b = lax.bitcast_convert_type(sc, jnp.int32)
    m = b ^ (lax.shift_right_arithmetic(b, 31) & jnp.int32(0x7FFFFFFF))

    def tbody(i, t):
        cand = t | lax.shift_left(jnp.int32(1), 31 - i)
        thr_s = cand ^ jnp.int32(_MIN32)
        cnt = jnp.sum((m >= thr_s).astype(jnp.int32))
        return jnp.where(cnt >= K, cand, t)

    T = lax.fori_loop(0, 32, tbody, jnp.int32(0))
    ts = T ^ jnp.int32(_MIN32)
    c_gt = jnp.sum((m > ts).astype(jnp.int32))
    need = K - c_gt
    ties = m == ts

    def jbody(i, jv):
        cand = jv | lax.shift_left(jnp.int32(1), 13 - i)
        cnt = jnp.sum((ties & (r < cand)).astype(jnp.int32))
        return jnp.where(cnt < need, cand, jv)

    J = lax.fori_loop(0, 14, jbody, jnp.int32(0))
    sel = (m > ts) | (ties & (r <= J) & (need > 0))
    g = jnp.tanh(sc)
    w_ref[...] = jnp.where(sel, g, 0.0)
    m_ref[...] = jnp.where(sel, 1.0, 0.0)


def _e0_call(sp, si80, bs11):
    return pl.pallas_call(
        _e0_body,
        out_shape=[jax.ShapeDtypeStruct((NROW, 128), jnp.float32),
                   jax.ShapeDtypeStruct((NROW, 128), jnp.float32)],
    )(sp, si80, bs11)


def _e1_body(w_ref, m_ref, cat_ref, wl1_ref, bl1_ref, wl2_ref, bl2_ref,
             wl3_ref, bl3_ref, out_ref, sum_acc, max_acc):
    b = pl.program_id(0)

    @pl.when(b < _NB)
    def _():
        gated = jnp.where(m_ref[...] > 0.0, cat_ref[...] * w_ref[...], 0.0)
        psum = jnp.sum(gated, axis=0, keepdims=True)
        pmax = jnp.max(jnp.where(m_ref[...] > 0.0, gated, -jnp.inf),
                       axis=0, keepdims=True)

        @pl.when(b == 0)
        def _():
            sum_acc[...] = psum
            max_acc[...] = pmax

        @pl.when(b > 0)
        def _():
            sum_acc[...] = sum_acc[...] + psum
            max_acc[...] = jnp.maximum(max_acc[...], pmax)

    @pl.when(b == _NB)
    def _():
        mean = sum_acc[...] * (1.0 / K)
        ro = jnp.concatenate([mean, max_acc[...]], axis=1)
        h = jnp.maximum(
            jnp.dot(ro, wl1_ref[...], preferred_element_type=jnp.float32)
            + bl1_ref[...], 0.0)
        h = jnp.maximum(
            jnp.dot(h, wl2_ref[...], preferred_element_type=jnp.float32)
            + bl2_ref[...], 0.0)
        lg = (jnp.dot(h, wl3_ref[...], preferred_element_type=jnp.float32)
              + bl3_ref[...])
        mx = jnp.max(lg, axis=1, keepdims=True)
        lse = jnp.log(jnp.sum(jnp.exp(lg - mx), axis=1, keepdims=True)) + mx
        out_ref[...] = lg - lse


def _e1_call(w_col, m_col, cat, Wl1, bl1, Wl2, bl2, Wl3, bl3):
    clamp = lambda b: (jnp.minimum(b, _NB - 1), 0)
    return pl.pallas_call(
        _e1_body,
        grid=(_NB + 1,),
        in_specs=[pl.BlockSpec((_BR, 1), clamp),
                  pl.BlockSpec((_BR, 1), clamp),
                  pl.BlockSpec((_BR, 3 * HID), clamp),
                  pl.BlockSpec((2 * 3 * HID, HID), lambda b: (0, 0)),
                  pl.BlockSpec((1, HID), lambda b: (0, 0)),
                  pl.BlockSpec((HID, HID // 2), lambda b: (0, 0)),
                  pl.BlockSpec((1, HID // 2), lambda b: (0, 0)),
                  pl.BlockSpec((HID // 2, 10), lambda b: (0, 0)),
                  pl.BlockSpec((1, 10), lambda b: (0, 0))],
        out_specs=pl.BlockSpec((1, 10), lambda b: (0, 0)),
        out_shape=jax.ShapeDtypeStruct((1, 10), jnp.float32),
        scratch_shapes=[pltpu.VMEM((1, 3 * HID), jnp.float32),
                        pltpu.VMEM((1, 3 * HID), jnp.float32)],
    )(w_col, m_col, cat, Wl1, bl1, Wl2, bl2, Wl3, bl3)


def kernel(x, edge_index, W0, b0, W1, b1, W2, b2, Ws, bs,
           Wl1, bl1, Wl2, bl2, Wl3, bl3):
    src = edge_index[0]
    dst = edge_index[1]
    pad = EPAD - E
    fill = jnp.full((pad,), N, jnp.int32)
    srcp = jnp.concatenate([src, fill]).reshape(NW, NCH, CH)
    dstp = jnp.concatenate([dst, fill]).reshape(NW, NCH, CH)
    xp = jnp.concatenate([x, jnp.zeros((NP - N, D), x.dtype)], axis=0)

    comb_flat = src | (dst << 16)
    padv = N | (N << 16)
    c0 = NS * ACH * CH
    cap1 = NS * BCH * CH
    e0 = comb_flat[:c0].reshape(NS, ACH, CH)
    e1 = jnp.concatenate(
        [comb_flat[c0:], jnp.full((c0 + cap1 - E,), padv, jnp.int32)]
    ).reshape(NS, BCH, CH)
    e1f = jnp.concatenate(
        [e1, jnp.full((NS, ACH - BCH, CH), padv, jnp.int32)], axis=1)
    combw = jnp.stack([e0, e1f], axis=1).reshape(NW, ACH, CH)
    ztab = jnp.zeros((RPA, D), jnp.float32)

    degp = _deg_call(srcp, dstp)
    so80, si80 = _prep_call(degp.reshape(NC, 2, NROW, 128))
    so_col = so80.reshape(NP, 1)
    si_col = si80.reshape(NP, 1)

    xn = _scale_call(xp, so_col)
    p1 = _vecagg_call(xn, combw, ztab)
    y1n = _mid_call(p1, si_col, so_col)
    p2 = _vecagg_call(y1n, combw, ztab)
    y2n = _mid_call(p2, si_col, so_col)
    p3 = _vecagg_call(y2n, combw, ztab)

    W01, W012 = _wts_call(W0, W1, W2)
    cat, s0n = _feats_call(
        p1, p2, p3, si_col, so_col, W0, W01, W012,
        b0.reshape(1, HID), b1.reshape(1, HID), b2.reshape(1, HID),
        Ws[0:HID], Ws[HID:2 * HID], Ws[2 * HID:3 * HID])

    sp = _scalagg_call(s0n.reshape(NP), srcp, dstp)
    w80, m80 = _e0_call(sp.reshape(NC, NROW, 128), si80, bs.reshape(1, 1))
    return _e1_call(w80.reshape(NP, 1), m80.reshape(NP, 1), cat,
                    Wl1, bl1.reshape(1, HID), Wl2, bl2.reshape(1, HID // 2),
                    Wl3, bl3.reshape(1, 10))
